# Initial kernel scaffold; baseline (speedup 1.0000x reference)
import jax
import jax.numpy as jnp
from jax.experimental import pallas as pl


def kernel(x, codebook, proj_kernel):
    raise NotImplementedError("write your pallas kernel here")


# speedup vs baseline: 1.2304x; 1.2304x over previous
import jax
import jax.numpy as jnp
from jax.experimental import pallas as pl
from jax.experimental.pallas import tpu as pltpu

NUM_EMBEDDINGS = 1024
EMBED_DIM = 64
BLK = 2048


def _l2n(v):
    return v * jax.lax.rsqrt((v * v).sum(axis=-1, keepdims=True) + 1e-12)


def _vq_body(x_ref, cb_ref, proj_ref, disc_ref, quant_ref, cbp_scr, cb2_scr):
    @pl.when(pl.program_id(0) == 0)
    def _():
        cbp = jax.lax.dot_general(
            cb_ref[...], proj_ref[...], (((1,), (0,)), ((), ())),
            preferred_element_type=jnp.float32)
        cbp = _l2n(cbp)
        cbp_scr[...] = cbp
        cb2_scr[...] = (cbp * cbp).sum(axis=1, keepdims=True).reshape(1, -1)

    xp = jax.lax.dot_general(
        x_ref[...], proj_ref[...], (((1,), (0,)), ((), ())),
        preferred_element_type=jnp.float32)
    xp = _l2n(xp)
    x2 = (xp * xp).sum(axis=1, keepdims=True)
    dots = jax.lax.dot_general(
        xp, cbp_scr[...], (((1,), (1,)), ((), ())),
        preferred_element_type=jnp.float32)
    d = (x2 + (-2.0) * dots) + cb2_scr[...]
    idx = jnp.argmin(d, axis=1)
    disc = (jax.lax.broadcasted_iota(jnp.int32, d.shape, 1)
            == idx[:, None]).astype(jnp.float32)
    disc_ref[...] = disc
    q = jax.lax.dot_general(
        disc, cb_ref[...], (((1,), (0,)), ((), ())),
        preferred_element_type=jnp.float32)
    quant_ref[...] = _l2n(q)


def kernel(x, codebook, proj_kernel):
    x_flat = x.reshape(-1, EMBED_DIM)
    n = x_flat.shape[0]
    grid = n // BLK
    disc, quant = pl.pallas_call(
        _vq_body,
        grid=(grid,),
        in_specs=[
            pl.BlockSpec((BLK, EMBED_DIM), lambda i: (i, 0)),
            pl.BlockSpec((NUM_EMBEDDINGS, EMBED_DIM), lambda i: (0, 0)),
            pl.BlockSpec((EMBED_DIM, EMBED_DIM), lambda i: (0, 0)),
        ],
        out_specs=[
            pl.BlockSpec((BLK, NUM_EMBEDDINGS), lambda i: (i, 0)),
            pl.BlockSpec((BLK, EMBED_DIM), lambda i: (i, 0)),
        ],
        out_shape=[
            jax.ShapeDtypeStruct((n, NUM_EMBEDDINGS), jnp.float32),
            jax.ShapeDtypeStruct((n, EMBED_DIM), jnp.float32),
        ],
        scratch_shapes=[
            pltpu.VMEM((NUM_EMBEDDINGS, EMBED_DIM), jnp.float32),
            pltpu.VMEM((1, NUM_EMBEDDINGS), jnp.float32),
        ],
    )(x_flat, codebook, proj_kernel)
    return disc, quant.reshape(x.shape[:-1] + (EMBED_DIM,))
